# Initial kernel scaffold; baseline (speedup 1.0000x reference)
import jax
import jax.numpy as jnp
from jax.experimental import pallas as pl


def kernel(x, edge_index, batch, W1_rel, b1_rel, W1_root, W5_rel, b5_rel, W5_root, Wl1, bl1, Wl2, bl2, Wl3, bl3, Wl4, bl4):
    raise NotImplementedError("write your pallas kernel here")


# speedup vs baseline: 56.9434x; 56.9434x over previous
import functools

import jax
import jax.numpy as jnp
from jax import lax
from jax.experimental import pallas as pl
from jax.experimental.pallas import tpu as pltpu
from jax.experimental.pallas import tpu_sc as plsc

N = 50000
E = 800000
HID = 64
NGRAPH = 4
NP = 51200
EP = 819200
NWORKERS = 32
EDGES_PER_TILE = EP // NWORKERS
CHUNK = 5120
NCHUNK = EDGES_PER_TILE // CHUNK
STREAM = 128
NSTREAM = CHUNK // STREAM
ZSLICE = NP // 16


def _segsum_body(vals_hbm, src_hbm, dst_hbm, zeros_hbm, out_hbm,
                 vals_v, src_v, gath_v, dst_v, acc_sh, sem):
    cid = lax.axis_index("c")
    sid = lax.axis_index("s")
    wid = sid * 2 + cid

    pltpu.sync_copy(zeros_hbm.at[pl.ds(sid * ZSLICE, ZSLICE)],
                    acc_sh.at[pl.ds(sid * ZSLICE, ZSLICE)])
    pltpu.sync_copy(vals_hbm, vals_v)
    plsc.subcore_barrier()

    for c in range(NCHUNK):
        base = pl.multiple_of(wid * EDGES_PER_TILE + c * CHUNK, CHUNK)
        row = pl.multiple_of(wid * (EDGES_PER_TILE // 128) + c * NSTREAM, 8)
        pltpu.sync_copy(src_hbm.at[pl.ds(base, CHUNK)], src_v)
        pltpu.sync_copy(dst_hbm.at[pl.ds(row, NSTREAM)], dst_v)

        def gbody(i, carry):
            s = src_v[pl.ds(i * 16, 16)]
            gath_v[pl.ds(i * 16, 16)] = plsc.load_gather(vals_v, [s])
            return carry
        lax.fori_loop(0, CHUNK // 16, gbody, 0)

        copies = [
            pltpu.async_copy(gath_v.at[pl.ds(j * STREAM, STREAM)],
                             acc_sh.at[dst_v.at[j]], sem, add=True)
            for j in range(NSTREAM)
        ]
        for cp in copies:
            cp.wait()

    plsc.subcore_barrier()
    pltpu.sync_copy(acc_sh.at[pl.ds(sid * ZSLICE, ZSLICE)],
                    out_hbm.at[cid, pl.ds(sid * ZSLICE, ZSLICE)])


_segsum = pl.kernel(
    _segsum_body,
    out_type=jax.ShapeDtypeStruct((2, NP), jnp.float32),
    mesh=plsc.VectorSubcoreMesh(core_axis_name="c", subcore_axis_name="s"),
    scratch_types=[
        pltpu.VMEM((NP,), jnp.float32),
        pltpu.VMEM((CHUNK,), jnp.int32),
        pltpu.VMEM((CHUNK,), jnp.float32),
        pltpu.VMEM((NSTREAM, STREAM), jnp.int32),
        pltpu.VMEM_SHARED((NP,), jnp.float32),
        pltpu.SemaphoreType.DMA,
    ],
    compiler_params=pltpu.CompilerParams(needs_layout_passes=False),
)


def _pq_body(agg_ref, x_ref, wr_ref, wt_ref, b1_ref, w5r_ref, w5t_ref,
             b5_ref, p_ref, q_ref):
    a = agg_ref[0] + agg_ref[1]
    xv = x_ref[...]
    p = jnp.zeros_like(xv)
    q = jnp.zeros_like(xv)
    for k in range(HID):
        t = jnp.maximum(a * wr_ref[k] + xv * wt_ref[k] + b1_ref[k], 0.0)
        p = p + t * w5r_ref[k]
        q = q + t * w5t_ref[k]
    p_ref[...] = p
    q_ref[...] = q + b5_ref[0]


def _pq(agg, xp, wr, wt, b1, w5r, w5t, b5):
    smem = pl.BlockSpec(memory_space=pltpu.SMEM)
    return pl.pallas_call(
        _pq_body,
        out_shape=[jax.ShapeDtypeStruct((NP // 128, 128), jnp.float32)] * 2,
        in_specs=[pl.BlockSpec((2, NP // 128, 128), lambda: (0, 0, 0)),
                  pl.BlockSpec((NP // 128, 128), lambda: (0, 0)),
                  smem, smem, smem, smem, smem, smem],
    )(agg, xp, wr, wt, b1, w5r, w5t, b5)


def _assemble_body(agg_ref, q_ref, batch_ref, out_ref):
    b = batch_ref[...]
    bs = jnp.float32(0)
    for g in range(NGRAPH):
        bs = bs + jnp.max(jnp.where(b == g, 1.0, 0.0))
    out_ref[...] = agg_ref[0] + agg_ref[1] + q_ref[...] + (bs - NGRAPH)


def _assemble(agg2, q, batch_p):
    return pl.pallas_call(
        _assemble_body,
        out_shape=jax.ShapeDtypeStruct((NP // 128, 128), jnp.float32),
    )(agg2, q, batch_p)


def _mm1_body(h_ref, w_ref, out_ref):
    out_ref[...] = lax.dot_general(
        h_ref[...], w_ref[...], (((1,), (1,)), ((), ())),
        preferred_element_type=jnp.float32)


def _mm1(hmlp, Wl1):
    blk = 512
    grid = (pl.cdiv(Wl1.shape[0], blk),)
    return pl.pallas_call(
        _mm1_body,
        grid=grid,
        in_specs=[pl.BlockSpec((NGRAPH, N // NGRAPH), lambda i: (0, 0)),
                  pl.BlockSpec((blk, N // NGRAPH), lambda i: (i, 0))],
        out_specs=pl.BlockSpec((NGRAPH, blk), lambda i: (0, i)),
        out_shape=jax.ShapeDtypeStruct((NGRAPH, Wl1.shape[0]), jnp.float32),
    )(hmlp, Wl1)


def _tail_body(z1_ref, bl1_ref, w2_ref, bl2_ref, w3_ref, bl3_ref,
               w4_ref, bl4_ref, out_ref):
    y = jnp.maximum(z1_ref[...] + bl1_ref[...], 0.0)
    y = lax.dot_general(y, w2_ref[...], (((1,), (1,)), ((), ())),
                        preferred_element_type=jnp.float32)
    y = jnp.maximum(y + bl2_ref[...], 0.0)
    y = lax.dot_general(y, w3_ref[...], (((1,), (1,)), ((), ())),
                        preferred_element_type=jnp.float32)
    y = jnp.maximum(y + bl3_ref[...], 0.0)
    y = lax.dot_general(y, w4_ref[...], (((1,), (1,)), ((), ())),
                        preferred_element_type=jnp.float32)
    y = y + bl4_ref[...]
    m = jnp.max(y, axis=1, keepdims=True)
    lse = jnp.log(jnp.sum(jnp.exp(y - m), axis=1, keepdims=True)) + m
    out_ref[...] = y - lse


def _tail(z1, bl1, Wl2, bl2, Wl3, bl3, Wl4, bl4):
    return pl.pallas_call(
        _tail_body,
        out_shape=jax.ShapeDtypeStruct((NGRAPH, 10), jnp.float32),
    )(z1, bl1, Wl2, bl2, Wl3, bl3, Wl4, bl4)


def kernel(x, edge_index, batch, W1_rel, b1_rel, W1_root, W5_rel, b5_rel,
           W5_root, Wl1, bl1, Wl2, bl2, Wl3, bl3, Wl4, bl4):
    xf = x[:, 0]
    src = edge_index[0]
    dst = edge_index[1]
    padE = EP - E
    pad_idx = jnp.arange(padE, dtype=jnp.int32)
    src_p = jnp.concatenate([src, pad_idx % N])
    dst_p = jnp.concatenate([dst, N + pad_idx % (NP - N)]).reshape(EP // 128, 128)
    vals1 = jnp.pad(xf, (0, NP - N))
    zeros = jnp.zeros((NP,), jnp.float32)

    agg1 = _segsum(vals1, src_p, dst_p, zeros)
    p, q = _pq(agg1.reshape(2, NP // 128, 128), vals1.reshape(NP // 128, 128),
               W1_rel[:, 0], W1_root[:, 0], b1_rel, W5_rel[0], W5_root[0],
               b5_rel)
    agg2 = _segsum(p.reshape(NP), src_p, dst_p, zeros)

    batch_p = jnp.concatenate(
        [batch, jnp.broadcast_to(batch[-1], (NP - N,))]).reshape(NP // 128, 128)
    out2 = _assemble(agg2.reshape(2, NP // 128, 128), q, batch_p)
    hmlp = out2.reshape(NP)[:N].reshape(NGRAPH, N // NGRAPH)

    z1 = _mm1(hmlp, Wl1)
    return _tail(z1, bl1.reshape(1, -1), Wl2, bl2.reshape(1, -1),
                 Wl3, bl3.reshape(1, -1), Wl4, bl4.reshape(1, -1))
